# baseline (speedup 1.0000x reference)
import functools

import jax
import jax.numpy as jnp
from jax import lax
from jax.experimental import pallas as pl
from jax.experimental.pallas import tpu as pltpu
from jax.experimental.pallas import tpu_sc as plsc

BN_EPS = 1e-5

_NC = 2
_NS = 16
_NW = _NC * _NS
_K = 80


def _sc_agg(h, packed, n):
    d = h.shape[1]
    e = packed.shape[0]
    ew = e // _NW
    nch = ew // _K
    rt = (n // _NS) // 8 * 8
    tail = n - rt * _NS

    nb = 2

    mesh = plsc.VectorSubcoreMesh(core_axis_name="c", subcore_axis_name="s")

    @functools.partial(
        pl.kernel,
        out_type=(
            jax.ShapeDtypeStruct((n, d), jnp.float32),
            jax.ShapeDtypeStruct((n, d), jnp.float32),
        ),
        mesh=mesh,
        scratch_types=[
            pltpu.VMEM_SHARED((n, d), jnp.float32),
            pltpu.VMEM((ew,), jnp.int32),
            pltpu.VMEM((nb, _K), jnp.int32),
            pltpu.VMEM((nb, _K), jnp.int32),
            pltpu.VMEM((nb, _K, d), jnp.float32),
        ] + [pltpu.SemaphoreType.DMA] * (2 * nb),
    )
    def agg_kernel(h_hbm, pk_hbm, zeros_hbm, out0, out1,
                   accs, pslab, sring, dring, bufs, *sems):
        gsems, ssems = sems[:nb], sems[nb:]
        c = lax.axis_index("c")
        s = lax.axis_index("s")
        wid = c * _NS + s
        base0 = wid * ew

        pltpu.sync_copy(pk_hbm.at[pl.ds(base0, ew)], pslab)

        def _unpack(ci, b):
            def _vec(j, _):
                pk = pslab[pl.ds(ci * _K + j * 16, 16)]
                sring[b, pl.ds(j * 16, 16)] = lax.shift_right_logical(pk, 14)
                dring[b, pl.ds(j * 16, 16)] = lax.bitwise_and(pk, 16383)
                return 0
            lax.fori_loop(0, _K // 16, _vec, 0)

        def _gather_start(ci, b):
            _unpack(ci, b)
            pltpu.async_copy(h_hbm.at[sring.at[b]], bufs.at[b], gsems[b])

        def _gather_wait(b):
            pltpu.make_async_copy(
                h_hbm.at[sring.at[b]], bufs.at[b], gsems[b]).wait()

        def _scatter_start(b):
            pltpu.async_copy(bufs.at[b], accs.at[dring.at[b]], ssems[b],
                             add=True)

        def _scatter_wait(b):
            pltpu.make_async_copy(bufs.at[b], accs.at[dring.at[b]],
                                  ssems[b]).wait()

        for b in range(nb):
            _gather_start(b, b)

        pltpu.sync_copy(zeros_hbm.at[pl.ds(0, rt)], accs.at[pl.ds(s * rt, rt)])

        @pl.when(s == 0)
        def _():
            pltpu.sync_copy(zeros_hbm.at[pl.ds(0, tail)],
                            accs.at[pl.ds(rt * _NS, tail)])
        plsc.subcore_barrier()

        def _body(g, _):
            for b in range(nb):
                _gather_wait(b)
                _scatter_start(b)
            for b in range(nb):
                ci = g * nb + b
                _scatter_wait(b)
                _gather_start(ci + nb, b)
            return 0
        lax.fori_loop(0, (nch - nb) // nb, _body, 0)
        rem = nch - nb - (nch - nb) // nb * nb
        for j in range(rem + nb):
            ci = (nch - nb - rem) + j
            b = ci % nb
            _gather_wait(b)
            _scatter_start(b)
            _scatter_wait(b)
            if j < rem:
                _gather_start(ci + nb, b)
        plsc.subcore_barrier()

        @pl.when(c == 0)
        def _():
            pltpu.sync_copy(accs.at[pl.ds(s * rt, rt)],
                            out0.at[pl.ds(s * rt, rt)])

            @pl.when(s == 0)
            def _():
                pltpu.sync_copy(accs.at[pl.ds(rt * _NS, tail)],
                                out0.at[pl.ds(rt * _NS, tail)])

        @pl.when(c == 1)
        def _():
            pltpu.sync_copy(accs.at[pl.ds(s * rt, rt)],
                            out1.at[pl.ds(s * rt, rt)])

            @pl.when(s == 0)
            def _():
                pltpu.sync_copy(accs.at[pl.ds(rt * _NS, tail)],
                                out1.at[pl.ds(rt * _NS, tail)])

    return agg_kernel(h, packed, jnp.zeros((rt, d), jnp.float32))


def _bn_relu(t, g, be):
    mu = jnp.mean(t, axis=0, keepdims=True)
    var = jnp.mean((t - mu) * (t - mu), axis=0, keepdims=True)
    return jnp.maximum(g * (t - mu) * lax.rsqrt(var + BN_EPS) + be, 0.0)


def _mlp_body(h_ref, a0_ref, a1_ref, eps_ref, w1_ref, b1_ref, g1_ref,
              be1_ref, w2_ref, b2_ref, g2_ref, be2_ref):
    z = (1.0 + eps_ref[0, 0]) * h_ref[...] + a0_ref[...] + a1_ref[...]
    t = jnp.dot(z, w1_ref[...], preferred_element_type=jnp.float32) + b1_ref[...]
    u = _bn_relu(t, g1_ref[...], be1_ref[...])
    v = jnp.dot(u, w2_ref[...], preferred_element_type=jnp.float32) + b2_ref[...]
    return _bn_relu(v, g2_ref[...], be2_ref[...])


def _tc_mlp(h, a0, a1, p):
    n, d = h.shape
    hdim = p["W1"].shape[1]

    def body(*refs):
        out_ref = refs[-1]
        out_ref[...] = _mlp_body(*refs[:-1])

    return pl.pallas_call(
        body,
        out_shape=jax.ShapeDtypeStruct((n, hdim), jnp.float32),
    )(h, a0, a1, p["eps"].reshape(1, 1),
      p["W1"], p["b1"].reshape(1, hdim), p["g1"].reshape(1, hdim),
      p["be1"].reshape(1, hdim),
      p["W2"], p["b2"].reshape(1, hdim), p["g2"].reshape(1, hdim),
      p["be2"].reshape(1, hdim))


def _tc_mlp_pool(h, a0, a1, p, batch2d, num_graphs):
    n, d = h.shape
    hdim = p["W1"].shape[1]

    def body(*refs):
        batch_ref, out_ref = refs[-2], refs[-1]
        w = _mlp_body(*refs[:-2])
        onehot = (batch_ref[...] ==
                  lax.broadcasted_iota(jnp.int32, (n, num_graphs), 1)
                  ).astype(jnp.float32)
        psum = lax.dot_general(onehot, w, (((0,), (0,)), ((), ())),
                               preferred_element_type=jnp.float32)
        ones = jnp.ones((n, 1), jnp.float32)
        cnt = lax.dot_general(onehot, ones, (((0,), (0,)), ((), ())),
                              preferred_element_type=jnp.float32)
        out_ref[...] = psum / jnp.maximum(cnt, 1.0)

    return pl.pallas_call(
        body,
        out_shape=jax.ShapeDtypeStruct((num_graphs, hdim), jnp.float32),
    )(h, a0, a1, p["eps"].reshape(1, 1),
      p["W1"], p["b1"].reshape(1, hdim), p["g1"].reshape(1, hdim),
      p["be1"].reshape(1, hdim),
      p["W2"], p["b2"].reshape(1, hdim), p["g2"].reshape(1, hdim),
      p["be2"].reshape(1, hdim), batch2d)


def kernel(x, edge_index, batch, params):
    n = x.shape[0]
    packed = jnp.bitwise_or(jnp.left_shift(edge_index[0], 14), edge_index[1])
    num_graphs = 64
    batch2d = batch.reshape(-1, 1)
    h = x
    for p in params[:-1]:
        a0, a1 = _sc_agg(h, packed, n)
        h = _tc_mlp(h, a0, a1, p)
    a0, a1 = _sc_agg(h, packed, n)
    return _tc_mlp_pool(h, a0, a1, params[-1], batch2d, num_graphs)


# speedup vs baseline: 1.4834x; 1.4834x over previous
import functools

import jax
import jax.numpy as jnp
from jax import lax
from jax.experimental import pallas as pl
from jax.experimental.pallas import tpu as pltpu
from jax.experimental.pallas import tpu_sc as plsc

BN_EPS = 1e-5

_NC = 2
_NS = 16
_NW = _NC * _NS
_K = 80


def _sc_agg(h, packed, n):
    d = h.shape[1]
    e = packed.shape[0]
    ew = e // _NW
    nch = ew // _K
    rt = (n // _NS) // 8 * 8
    tail = n - rt * _NS

    nb = 3

    mesh = plsc.VectorSubcoreMesh(core_axis_name="c", subcore_axis_name="s")

    @functools.partial(
        pl.kernel,
        out_type=(
            jax.ShapeDtypeStruct((n, d), jnp.float32),
            jax.ShapeDtypeStruct((n, d), jnp.float32),
        ),
        mesh=mesh,
        scratch_types=[
            pltpu.VMEM_SHARED((n, d), jnp.float32),
            pltpu.VMEM((ew,), jnp.int32),
            pltpu.VMEM((nb, _K), jnp.int32),
            pltpu.VMEM((nb, _K), jnp.int32),
            pltpu.VMEM((nb, _K, d), jnp.float32),
        ] + [pltpu.SemaphoreType.DMA] * (2 * nb),
    )
    def agg_kernel(h_hbm, pk_hbm, zeros_hbm, out0, out1,
                   accs, pslab, sring, dring, bufs, *sems):
        gsems, ssems = sems[:nb], sems[nb:]
        c = lax.axis_index("c")
        s = lax.axis_index("s")
        wid = c * _NS + s
        base0 = wid * ew

        pltpu.sync_copy(pk_hbm.at[pl.ds(base0, ew)], pslab)

        def _unpack(ci, b):
            def _vec(j, _):
                pk = pslab[pl.ds(ci * _K + j * 16, 16)]
                sring[b, pl.ds(j * 16, 16)] = lax.shift_right_logical(pk, 14)
                dring[b, pl.ds(j * 16, 16)] = lax.bitwise_and(pk, 16383)
                return 0
            lax.fori_loop(0, _K // 16, _vec, 0)

        def _gather_start(ci, b):
            _unpack(ci, b)
            pltpu.async_copy(h_hbm.at[sring.at[b]], bufs.at[b], gsems[b])

        def _gather_wait(b):
            pltpu.make_async_copy(
                h_hbm.at[sring.at[b]], bufs.at[b], gsems[b]).wait()

        def _scatter_start(b):
            pltpu.async_copy(bufs.at[b], accs.at[dring.at[b]], ssems[b],
                             add=True)

        def _scatter_wait(b):
            pltpu.make_async_copy(bufs.at[b], accs.at[dring.at[b]],
                                  ssems[b]).wait()

        for b in range(nb):
            _gather_start(b, b)

        pltpu.sync_copy(zeros_hbm.at[pl.ds(0, rt)], accs.at[pl.ds(s * rt, rt)])

        @pl.when(s == 0)
        def _():
            pltpu.sync_copy(zeros_hbm.at[pl.ds(0, tail)],
                            accs.at[pl.ds(rt * _NS, tail)])
        plsc.subcore_barrier()

        def _body(g, _):
            for b in range(nb):
                ci = g * nb + b
                _gather_wait(b)
                _scatter_start(b)
                _scatter_wait(b)
                _gather_start(ci + nb, b)
            return 0
        lax.fori_loop(0, (nch - nb) // nb, _body, 0)
        rem = nch - nb - (nch - nb) // nb * nb
        for j in range(rem + nb):
            ci = (nch - nb - rem) + j
            b = ci % nb
            _gather_wait(b)
            _scatter_start(b)
            _scatter_wait(b)
            if j < rem:
                _gather_start(ci + nb, b)
        plsc.subcore_barrier()

        @pl.when(c == 0)
        def _():
            pltpu.sync_copy(accs.at[pl.ds(s * rt, rt)],
                            out0.at[pl.ds(s * rt, rt)])

            @pl.when(s == 0)
            def _():
                pltpu.sync_copy(accs.at[pl.ds(rt * _NS, tail)],
                                out0.at[pl.ds(rt * _NS, tail)])

        @pl.when(c == 1)
        def _():
            pltpu.sync_copy(accs.at[pl.ds(s * rt, rt)],
                            out1.at[pl.ds(s * rt, rt)])

            @pl.when(s == 0)
            def _():
                pltpu.sync_copy(accs.at[pl.ds(rt * _NS, tail)],
                                out1.at[pl.ds(rt * _NS, tail)])

    return agg_kernel(h, packed, jnp.zeros((rt, d), jnp.float32))


def _bn_relu(t, g, be):
    mu = jnp.mean(t, axis=0, keepdims=True)
    var = jnp.mean((t - mu) * (t - mu), axis=0, keepdims=True)
    return jnp.maximum(g * (t - mu) * lax.rsqrt(var + BN_EPS) + be, 0.0)


def _mlp_body(h_ref, a0_ref, a1_ref, eps_ref, w1_ref, b1_ref, g1_ref,
              be1_ref, w2_ref, b2_ref, g2_ref, be2_ref):
    z = (1.0 + eps_ref[0, 0]) * h_ref[...] + a0_ref[...] + a1_ref[...]
    t = jnp.dot(z, w1_ref[...], preferred_element_type=jnp.float32) + b1_ref[...]
    u = _bn_relu(t, g1_ref[...], be1_ref[...])
    v = jnp.dot(u, w2_ref[...], preferred_element_type=jnp.float32) + b2_ref[...]
    return _bn_relu(v, g2_ref[...], be2_ref[...])


def _tc_mlp(h, a0, a1, p):
    n, d = h.shape
    hdim = p["W1"].shape[1]

    def body(*refs):
        out_ref = refs[-1]
        out_ref[...] = _mlp_body(*refs[:-1])

    return pl.pallas_call(
        body,
        out_shape=jax.ShapeDtypeStruct((n, hdim), jnp.float32),
    )(h, a0, a1, p["eps"].reshape(1, 1),
      p["W1"], p["b1"].reshape(1, hdim), p["g1"].reshape(1, hdim),
      p["be1"].reshape(1, hdim),
      p["W2"], p["b2"].reshape(1, hdim), p["g2"].reshape(1, hdim),
      p["be2"].reshape(1, hdim))


def _tc_mlp_pool(h, a0, a1, p, batch2d, num_graphs):
    n, d = h.shape
    hdim = p["W1"].shape[1]

    def body(*refs):
        batch_ref, out_ref = refs[-2], refs[-1]
        w = _mlp_body(*refs[:-2])
        onehot = (batch_ref[...] ==
                  lax.broadcasted_iota(jnp.int32, (n, num_graphs), 1)
                  ).astype(jnp.float32)
        psum = lax.dot_general(onehot, w, (((0,), (0,)), ((), ())),
                               preferred_element_type=jnp.float32)
        ones = jnp.ones((n, 1), jnp.float32)
        cnt = lax.dot_general(onehot, ones, (((0,), (0,)), ((), ())),
                              preferred_element_type=jnp.float32)
        out_ref[...] = psum / jnp.maximum(cnt, 1.0)

    return pl.pallas_call(
        body,
        out_shape=jax.ShapeDtypeStruct((num_graphs, hdim), jnp.float32),
    )(h, a0, a1, p["eps"].reshape(1, 1),
      p["W1"], p["b1"].reshape(1, hdim), p["g1"].reshape(1, hdim),
      p["be1"].reshape(1, hdim),
      p["W2"], p["b2"].reshape(1, hdim), p["g2"].reshape(1, hdim),
      p["be2"].reshape(1, hdim), batch2d)


def kernel(x, edge_index, batch, params):
    n = x.shape[0]
    packed = jnp.bitwise_or(jnp.left_shift(edge_index[0], 14), edge_index[1])
    num_graphs = 64
    batch2d = batch.reshape(-1, 1)
    h = x
    for p in params[:-1]:
        a0, a1 = _sc_agg(h, packed, n)
        h = _tc_mlp(h, a0, a1, p)
    a0, a1 = _sc_agg(h, packed, n)
    return _tc_mlp_pool(h, a0, a1, params[-1], batch2d, num_graphs)
